# baseline (speedup 1.0000x reference)
import functools

import jax
import jax.numpy as jnp
from jax import lax
from jax.experimental import pallas as pl
from jax.experimental.pallas import tpu as pltpu
from jax.experimental.pallas import tpu_sc as plsc

_NC, _NS, _L = 2, 16, 16
_NW = _NC * _NS
_TPAD = 1024


def _sqrt_tables_kernel(ab_ref, sq_ref):
    ab = ab_ref[...]
    sq_ref[0:1, :] = jnp.sqrt(ab)
    sq_ref[1:2, :] = jnp.sqrt(jnp.maximum(1.0 - ab, 0.0))


def _make_sc_kernel(B, SD):
    rows = B // _NW
    nchunk = SD // _L
    unroll = 8
    mesh = plsc.VectorSubcoreMesh(core_axis_name="c", subcore_axis_name="s")

    @functools.partial(
        pl.kernel,
        out_type=jax.ShapeDtypeStruct((B, SD), jnp.float32),
        mesh=mesh,
        scratch_types=[
            pltpu.VMEM((rows,), jnp.int32),
            pltpu.VMEM((_TPAD,), jnp.float32),
            pltpu.VMEM((_TPAD,), jnp.float32),
            pltpu.VMEM((2, SD), jnp.float32),
            pltpu.VMEM((2, SD), jnp.float32),
            pltpu.VMEM((2, SD), jnp.float32),
            pltpu.SemaphoreType.DMA((2,)),
            pltpu.SemaphoreType.DMA((2,)),
            pltpu.SemaphoreType.DMA((2,)),
        ],
        compiler_params=pltpu.CompilerParams(needs_layout_passes=False),
    )
    def sc_kernel(sqa_hbm, sqb_hbm, t_hbm, x_hbm, e_hbm, o_hbm,
                  tv, sqa_v, sqb_v, xb, eb, ob, sx, se, so):
        wid = lax.axis_index("s") * _NC + lax.axis_index("c")
        base = wid * rows

        pltpu.sync_copy(sqa_hbm, sqa_v)
        pltpu.sync_copy(sqb_hbm, sqb_v)
        pltpu.sync_copy(t_hbm.at[pl.ds(base, rows)], tv)

        def in_x(r, slot):
            return pltpu.make_async_copy(x_hbm.at[base + r], xb.at[slot],
                                         sx.at[slot])

        def in_e(r, slot):
            return pltpu.make_async_copy(e_hbm.at[base + r], eb.at[slot],
                                         se.at[slot])

        def out_o(r, slot):
            return pltpu.make_async_copy(ob.at[slot], o_hbm.at[base + r],
                                         so.at[slot])

        for s in range(2):
            in_x(s, s).start()
            in_e(s, s).start()

        def body(r, _):
            slot = lax.rem(r, 2)
            in_x(r, slot).wait()
            in_e(r, slot).wait()

            ridx = jnp.full((_L,), r, dtype=jnp.int32)
            tr = plsc.load_gather(tv, [ridx])
            av = plsc.load_gather(sqa_v, [tr])
            bv = plsc.load_gather(sqb_v, [tr])

            @pl.when(r >= 2)
            def _():
                out_o(r - 2, slot).wait()

            def chunk(j, _):
                for u in range(unroll):
                    k = j * unroll + u
                    ob[slot, pl.ds(k * _L, _L)] = (
                        av * xb[slot, pl.ds(k * _L, _L)]
                        + bv * eb[slot, pl.ds(k * _L, _L)]
                    )
                return 0

            lax.fori_loop(0, nchunk // unroll, chunk, 0)
            out_o(r, slot).start()

            @pl.when(r + 2 < rows)
            def _():
                in_x(r + 2, slot).start()
                in_e(r + 2, slot).start()

            return 0

        lax.fori_loop(0, rows, body, 0)
        out_o(rows - 2, lax.rem(rows - 2, 2)).wait()
        out_o(rows - 1, lax.rem(rows - 1, 2)).wait()

    return sc_kernel


def kernel(x0, t, eps, alphabar):
    B, S, D = x0.shape
    SD = S * D
    T = alphabar.shape[0]
    x2 = x0.reshape(B, SD)
    e2 = eps.reshape(B, SD)
    ti = t.astype(jnp.int32)
    abp = jnp.concatenate(
        [alphabar, jnp.full((_TPAD - T,), 0.5, jnp.float32)]
    ).reshape(1, _TPAD)
    sq = pl.pallas_call(
        _sqrt_tables_kernel,
        out_shape=jax.ShapeDtypeStruct((2, _TPAD), jnp.float32),
    )(abp)
    sqa = sq[0]
    sqb = sq[1]
    noisy2 = _make_sc_kernel(B, SD)(sqa, sqb, ti, x2, e2)
    return noisy2.reshape(B, S, D), eps


# speedup vs baseline: 1.5605x; 1.5605x over previous
import functools

import jax
import jax.numpy as jnp
from jax import lax
from jax.experimental import pallas as pl
from jax.experimental.pallas import tpu as pltpu
from jax.experimental.pallas import tpu_sc as plsc

_NC, _NS, _L = 2, 16, 16
_NW = _NC * _NS
_TPAD = 1024


def _sqrt_tables_kernel(ab_ref, sq_ref):
    ab = ab_ref[...]
    sq_ref[0:1, :] = jnp.sqrt(ab)
    sq_ref[1:2, :] = jnp.sqrt(jnp.maximum(1.0 - ab, 0.0))


def _make_sc_kernel(B, SD):
    rows = B // _NW
    nchunk = SD // _L
    unroll = 8
    mesh = plsc.VectorSubcoreMesh(core_axis_name="c", subcore_axis_name="s")

    @functools.partial(
        pl.kernel,
        out_type=jax.ShapeDtypeStruct((B, SD), jnp.float32),
        mesh=mesh,
        scratch_types=[
            pltpu.VMEM((rows,), jnp.int32),
            pltpu.VMEM((_TPAD,), jnp.float32),
            pltpu.VMEM((_TPAD,), jnp.float32),
            pltpu.VMEM((2, SD), jnp.float32),
            pltpu.VMEM((2, SD), jnp.float32),
            pltpu.VMEM((2, SD), jnp.float32),
            pltpu.SemaphoreType.DMA((2,)),
            pltpu.SemaphoreType.DMA((2,)),
            pltpu.SemaphoreType.DMA((2,)),
        ],
        compiler_params=pltpu.CompilerParams(needs_layout_passes=False),
    )
    def sc_kernel(sqa_hbm, sqb_hbm, t_hbm, x_hbm, e_hbm, o_hbm,
                  tv, sqa_v, sqb_v, xb, eb, ob, sx, se, so):
        wid = lax.axis_index("s") * _NC + lax.axis_index("c")
        base = wid * rows

        pltpu.sync_copy(sqa_hbm, sqa_v)
        pltpu.sync_copy(sqb_hbm, sqb_v)
        pltpu.sync_copy(t_hbm.at[pl.ds(base, rows)], tv)

        def in_x(r, slot):
            return pltpu.make_async_copy(x_hbm.at[base + r], xb.at[slot],
                                         sx.at[slot])

        def in_e(r, slot):
            return pltpu.make_async_copy(e_hbm.at[base + r], eb.at[slot],
                                         se.at[slot])

        def out_o(r, slot):
            return pltpu.make_async_copy(ob.at[slot], o_hbm.at[base + r],
                                         so.at[slot])

        for s in range(2):
            in_x(s, s).start()
            in_e(s, s).start()

        def body(r, _):
            slot = lax.rem(r, 2)
            in_x(r, slot).wait()
            in_e(r, slot).wait()

            ridx = jnp.full((_L,), r, dtype=jnp.int32)
            tr = plsc.load_gather(tv, [ridx])
            av = plsc.load_gather(sqa_v, [tr])
            bv = plsc.load_gather(sqb_v, [tr])

            @pl.when(r >= 2)
            def _():
                out_o(r - 2, slot).wait()

            @plsc.parallel_loop(0, nchunk, unroll=unroll)
            def _chunk(k):
                ob[slot, pl.ds(k * _L, _L)] = (
                    av * xb[slot, pl.ds(k * _L, _L)]
                    + bv * eb[slot, pl.ds(k * _L, _L)]
                )
            out_o(r, slot).start()

            @pl.when(r + 2 < rows)
            def _():
                in_x(r + 2, slot).start()
                in_e(r + 2, slot).start()

            return 0

        lax.fori_loop(0, rows, body, 0)
        out_o(rows - 2, lax.rem(rows - 2, 2)).wait()
        out_o(rows - 1, lax.rem(rows - 1, 2)).wait()

    return sc_kernel


def kernel(x0, t, eps, alphabar):
    B, S, D = x0.shape
    SD = S * D
    T = alphabar.shape[0]
    x2 = x0.reshape(B, SD)
    e2 = eps.reshape(B, SD)
    ti = t.astype(jnp.int32)
    abp = jnp.concatenate(
        [alphabar, jnp.full((_TPAD - T,), 0.5, jnp.float32)]
    ).reshape(1, _TPAD)
    sq = pl.pallas_call(
        _sqrt_tables_kernel,
        out_shape=jax.ShapeDtypeStruct((2, _TPAD), jnp.float32),
    )(abp)
    sqa = sq[0]
    sqb = sq[1]
    noisy2 = _make_sc_kernel(B, SD)(sqa, sqb, ti, x2, e2)
    return noisy2.reshape(B, S, D), eps


# speedup vs baseline: 1.5609x; 1.0003x over previous
import functools

import jax
import jax.numpy as jnp
from jax import lax
from jax.experimental import pallas as pl
from jax.experimental.pallas import tpu as pltpu
from jax.experimental.pallas import tpu_sc as plsc

_NC, _NS, _L = 2, 16, 16
_NW = _NC * _NS
_TPAD = 1024


def _sqrt_tables_kernel(ab_ref, sq_ref):
    ab = ab_ref[...]
    sq_ref[0:1, :] = jnp.sqrt(ab)
    sq_ref[1:2, :] = jnp.sqrt(jnp.maximum(1.0 - ab, 0.0))


def _make_sc_kernel(B, SD):
    rows = B // _NW
    nchunk = SD // _L
    unroll = 16
    mesh = plsc.VectorSubcoreMesh(core_axis_name="c", subcore_axis_name="s")

    @functools.partial(
        pl.kernel,
        out_type=jax.ShapeDtypeStruct((B, SD), jnp.float32),
        mesh=mesh,
        scratch_types=[
            pltpu.VMEM((rows,), jnp.int32),
            pltpu.VMEM((_TPAD,), jnp.float32),
            pltpu.VMEM((_TPAD,), jnp.float32),
            pltpu.VMEM((2, SD), jnp.float32),
            pltpu.VMEM((2, SD), jnp.float32),
            pltpu.VMEM((2, SD), jnp.float32),
            pltpu.SemaphoreType.DMA((2,)),
            pltpu.SemaphoreType.DMA((2,)),
            pltpu.SemaphoreType.DMA((2,)),
        ],
        compiler_params=pltpu.CompilerParams(needs_layout_passes=False),
    )
    def sc_kernel(sqa_hbm, sqb_hbm, t_hbm, x_hbm, e_hbm, o_hbm,
                  tv, sqa_v, sqb_v, xb, eb, ob, sx, se, so):
        wid = lax.axis_index("s") * _NC + lax.axis_index("c")
        base = wid * rows

        pltpu.sync_copy(sqa_hbm, sqa_v)
        pltpu.sync_copy(sqb_hbm, sqb_v)
        pltpu.sync_copy(t_hbm.at[pl.ds(base, rows)], tv)

        def in_x(r, slot):
            return pltpu.make_async_copy(x_hbm.at[base + r], xb.at[slot],
                                         sx.at[slot])

        def in_e(r, slot):
            return pltpu.make_async_copy(e_hbm.at[base + r], eb.at[slot],
                                         se.at[slot])

        def out_o(r, slot):
            return pltpu.make_async_copy(ob.at[slot], o_hbm.at[base + r],
                                         so.at[slot])

        for s in range(2):
            in_x(s, s).start()
            in_e(s, s).start()

        def body(p, _):
            for slot in range(2):
                r = p * 2 + slot
                in_x(r, slot).wait()
                in_e(r, slot).wait()

                ridx = jnp.full((_L,), r, dtype=jnp.int32)
                tr = plsc.load_gather(tv, [ridx])
                av = plsc.load_gather(sqa_v, [tr])
                bv = plsc.load_gather(sqb_v, [tr])

                @pl.when(r >= 2)
                def _():
                    out_o(r - 2, slot).wait()

                @plsc.parallel_loop(0, nchunk, unroll=unroll)
                def _chunk(k):
                    ob[slot, pl.ds(k * _L, _L)] = (
                        av * xb[slot, pl.ds(k * _L, _L)]
                        + bv * eb[slot, pl.ds(k * _L, _L)]
                    )
                out_o(r, slot).start()

                @pl.when(r + 2 < rows)
                def _():
                    in_x(r + 2, slot).start()
                    in_e(r + 2, slot).start()

            return 0

        lax.fori_loop(0, rows // 2, body, 0)
        out_o(rows - 2, 0).wait()
        out_o(rows - 1, 1).wait()

    return sc_kernel


def kernel(x0, t, eps, alphabar):
    B, S, D = x0.shape
    SD = S * D
    T = alphabar.shape[0]
    x2 = x0.reshape(B, SD)
    e2 = eps.reshape(B, SD)
    ti = t.astype(jnp.int32)
    abp = jnp.concatenate(
        [alphabar, jnp.full((_TPAD - T,), 0.5, jnp.float32)]
    ).reshape(1, _TPAD)
    sq = pl.pallas_call(
        _sqrt_tables_kernel,
        out_shape=jax.ShapeDtypeStruct((2, _TPAD), jnp.float32),
    )(abp)
    sqa = sq[0]
    sqb = sq[1]
    noisy2 = _make_sc_kernel(B, SD)(sqa, sqb, ti, x2, e2)
    return noisy2.reshape(B, S, D), eps


# speedup vs baseline: 1.7654x; 1.1310x over previous
import functools

import jax
import jax.numpy as jnp
from jax import lax
from jax.experimental import pallas as pl
from jax.experimental.pallas import tpu as pltpu
from jax.experimental.pallas import tpu_sc as plsc

_NC, _NS, _L = 2, 16, 16
_NW = _NC * _NS
_TPAD = 1024
_BB = 64


def _sqrt_tables_kernel(ab_ref, sq_ref):
    ab = ab_ref[...]
    sq_ref[0:1, :] = jnp.sqrt(ab)
    sq_ref[1:2, :] = jnp.sqrt(jnp.maximum(1.0 - ab, 0.0))


def _make_sc_gather(B):
    rows = B // _NW
    mesh = plsc.VectorSubcoreMesh(core_axis_name="c", subcore_axis_name="s")

    @functools.partial(
        pl.kernel,
        out_type=[
            jax.ShapeDtypeStruct((B,), jnp.float32),
            jax.ShapeDtypeStruct((B,), jnp.float32),
        ],
        mesh=mesh,
        scratch_types=[
            pltpu.VMEM((rows,), jnp.int32),
            pltpu.VMEM((_TPAD,), jnp.float32),
            pltpu.VMEM((_TPAD,), jnp.float32),
            pltpu.VMEM((rows,), jnp.float32),
            pltpu.VMEM((rows,), jnp.float32),
        ],
        compiler_params=pltpu.CompilerParams(needs_layout_passes=False),
    )
    def sc_gather(sqa_hbm, sqb_hbm, t_hbm, ca_hbm, cb_hbm,
                  tv, sqa_v, sqb_v, cav, cbv):
        wid = lax.axis_index("s") * _NC + lax.axis_index("c")
        base = wid * rows

        pltpu.sync_copy(sqa_hbm, sqa_v)
        pltpu.sync_copy(sqb_hbm, sqb_v)
        pltpu.sync_copy(t_hbm.at[pl.ds(base, rows)], tv)

        for k in range(rows // _L):
            idx = tv[pl.ds(k * _L, _L)]
            cav[pl.ds(k * _L, _L)] = plsc.load_gather(sqa_v, [idx])
            cbv[pl.ds(k * _L, _L)] = plsc.load_gather(sqb_v, [idx])

        pltpu.sync_copy(cav, ca_hbm.at[pl.ds(base, rows)])
        pltpu.sync_copy(cbv, cb_hbm.at[pl.ds(base, rows)])

    return sc_gather


def _scale_kernel(a_ref, b_ref, x0_ref, eps_ref, noisy_ref):
    noisy_ref[...] = a_ref[...] * x0_ref[...] + b_ref[...] * eps_ref[...]


def kernel(x0, t, eps, alphabar):
    B, S, D = x0.shape
    SD = S * D
    T = alphabar.shape[0]
    x2 = x0.reshape(B, SD)
    e2 = eps.reshape(B, SD)
    ti = t.astype(jnp.int32)
    abp = jnp.concatenate(
        [alphabar, jnp.full((_TPAD - T,), 0.5, jnp.float32)]
    ).reshape(1, _TPAD)
    sq = pl.pallas_call(
        _sqrt_tables_kernel,
        out_shape=jax.ShapeDtypeStruct((2, _TPAD), jnp.float32),
    )(abp)
    ca, cb = _make_sc_gather(B)(sq[0], sq[1], ti)
    ca2 = ca.reshape(B, 1)
    cb2 = cb.reshape(B, 1)
    grid = (B // _BB,)
    noisy = pl.pallas_call(
        _scale_kernel,
        grid=grid,
        in_specs=[
            pl.BlockSpec((_BB, 1), lambda i: (i, 0)),
            pl.BlockSpec((_BB, 1), lambda i: (i, 0)),
            pl.BlockSpec((_BB, SD), lambda i: (i, 0)),
            pl.BlockSpec((_BB, SD), lambda i: (i, 0)),
        ],
        out_specs=pl.BlockSpec((_BB, SD), lambda i: (i, 0)),
        out_shape=jax.ShapeDtypeStruct((B, SD), jnp.float32),
        compiler_params=pltpu.CompilerParams(
            dimension_semantics=("parallel",),
        ),
    )(ca2, cb2, x2, e2)
    return noisy.reshape(B, S, D), eps
